# baseline (speedup 1.0000x reference)
import functools

import jax
import jax.numpy as jnp
from jax import lax
from jax.experimental import pallas as pl
from jax.experimental.pallas import tpu as pltpu
from jax.experimental.pallas import tpu_sc as plsc

_NC = 2
_NS = 16
_NW = _NC * _NS
_L = 16


def _tc_add_kernel(x_ref, e_ref, o_ref):
    o_ref[...] = x_ref[...] + e_ref[...]


def _make_sc_kernel(B, PD, B1, S):
    RW = 8
    COLS = PD // 2
    NSEG = COLS // S
    mesh = plsc.VectorSubcoreMesh(core_axis_name="c", subcore_axis_name="s")

    @functools.partial(
        pl.kernel,
        mesh=mesh,
        out_type=jax.ShapeDtypeStruct((B - B1, PD), jnp.float32),
        scratch_types=[
            pltpu.VMEM((2, S), jnp.float32),
            pltpu.VMEM((2, RW, S), jnp.float32),
            pltpu.SemaphoreType.DMA,
            pltpu.SemaphoreType.DMA,
            pltpu.SemaphoreType.DMA,
        ],
    )
    def k(x_hbm, e_hbm, o_hbm, e_bufs, bufs, e_sem, in_sem, out_sem):
        wid = lax.axis_index("s") * _NC + lax.axis_index("c")
        og0 = (wid // 2) * RW
        col0 = (wid % 2) * COLS
        rows_in = pl.ds(B1 + og0, RW)
        rows_out = pl.ds(og0, RW)

        def e_copy(c, s):
            return pltpu.async_copy(
                e_hbm.at[pl.ds(col0 + c * S, S)], e_bufs.at[s], e_sem
            )

        def in_copy(c, s):
            return pltpu.async_copy(
                x_hbm.at[rows_in, pl.ds(col0 + c * S, S)], bufs.at[s], in_sem
            )

        def out_copy(c, s):
            return pltpu.async_copy(
                bufs.at[s], o_hbm.at[rows_out, pl.ds(col0 + c * S, S)], out_sem
            )

        def add_block(buf, e_v):
            def body(i, carry):
                sl = pl.ds(i * _L, _L)
                ev = e_v[sl]
                for r in range(RW):
                    plsc.addupdate(buf.at[r, sl], ev)
                return carry

            lax.fori_loop(0, S // _L, body, 0)

        e_d = [None] * NSEG
        in_d = [None] * NSEG
        out_d = [None] * NSEG
        e_d[0] = e_copy(0, 0)
        in_d[0] = in_copy(0, 0)
        for c in range(NSEG):
            s = c % 2
            if c + 1 < NSEG:
                if c >= 1:
                    out_d[c - 1].wait()
                e_d[c + 1] = e_copy(c + 1, 1 - s)
                in_d[c + 1] = in_copy(c + 1, 1 - s)
            e_d[c].wait()
            in_d[c].wait()
            add_block(bufs.at[s], e_bufs.at[s])
            out_d[c] = out_copy(c, s)
        out_d[NSEG - 2].wait()
        out_d[NSEG - 1].wait()

    return k


def kernel(encoded_patches, position_embedding):
    B, P, D = encoded_patches.shape
    PD = P * D
    x2 = encoded_patches.reshape(B, PD)
    e2 = position_embedding.reshape(1, PD)
    e1 = position_embedding.reshape(PD)
    B1 = B // 2
    BB = 16
    tc_half = pl.pallas_call(
        _tc_add_kernel,
        grid=(B1 // BB,),
        in_specs=[
            pl.BlockSpec((BB, PD), lambda i: (i, 0)),
            pl.BlockSpec((1, PD), lambda i: (0, 0)),
        ],
        out_specs=pl.BlockSpec((BB, PD), lambda i: (i, 0)),
        out_shape=jax.ShapeDtypeStruct((B1, PD), jnp.float32),
    )(x2, e2)
    sc_half = _make_sc_kernel(B, PD, B1, 3456)(x2, e1)
    out2 = jnp.concatenate([tc_half, sc_half], axis=0)
    return out2.reshape(B, P, D)


# speedup vs baseline: 1.2842x; 1.2842x over previous
import jax
import jax.numpy as jnp
from jax.experimental import pallas as pl
from jax.experimental.pallas import tpu as pltpu


def _make_stream_kernel(B, PD, CB, K, M):
    NCHUNK = B // CB

    def _stream_kernel(x_hbm, e_vmem, o_hbm, buf_in, buf_out, in_sem, out_sem):
        def in_copy(c):
            return pltpu.make_async_copy(
                x_hbm.at[pl.ds(c * CB, CB), :], buf_in.at[c % K], in_sem.at[c % K]
            )

        def out_copy(c):
            return pltpu.make_async_copy(
                buf_out.at[c % M], o_hbm.at[pl.ds(c * CB, CB), :], out_sem.at[c % M]
            )

        for c in range(min(K, NCHUNK)):
            in_copy(c).start()
        for c in range(NCHUNK):
            in_copy(c).wait()
            if c >= M:
                out_copy(c - M).wait()
            buf_out[c % M] = buf_in[c % K] + e_vmem[...]
            out_copy(c).start()
            if c + K < NCHUNK:
                in_copy(c + K).start()
        for c in range(max(NCHUNK - M, 0), NCHUNK):
            out_copy(c).wait()

    return _stream_kernel


def kernel(encoded_patches, position_embedding):
    B, P, D = encoded_patches.shape
    PD = P * D
    x2 = encoded_patches.reshape(B, PD)
    e2 = position_embedding.reshape(1, PD)
    CB = 1
    K = 16
    M = 16
    out2 = pl.pallas_call(
        _make_stream_kernel(B, PD, CB, K, M),
        in_specs=[
            pl.BlockSpec(memory_space=pltpu.MemorySpace.HBM),
            pl.BlockSpec(memory_space=pltpu.MemorySpace.VMEM),
        ],
        out_specs=pl.BlockSpec(memory_space=pltpu.MemorySpace.HBM),
        out_shape=jax.ShapeDtypeStruct((B, PD), jnp.float32),
        scratch_shapes=[
            pltpu.MemorySpace.VMEM((K, CB, PD), jnp.float32),
            pltpu.MemorySpace.VMEM((M, CB, PD), jnp.float32),
            pltpu.SemaphoreType.DMA((K,)),
            pltpu.SemaphoreType.DMA((M,)),
        ],
    )(x2, e2)
    return out2.reshape(B, P, D)
